# speedup vs baseline: 1.1599x; 1.1599x over previous
import functools

import jax
import jax.numpy as jnp
from jax import lax
from jax.experimental import pallas as pl
from jax.experimental.pallas import tpu as pltpu
from jax.experimental.pallas import tpu_sc as plsc

VOCAB = 100000
N_EMBD = 768
N_POS = 2048
BATCH = 4
SEQ = 2048

NTOK = BATCH * SEQ
NW = 32
ROWS_PER_W = SEQ // NW
LANES = 16
VECS_PER_ROW = N_EMBD // LANES


def _emb_body(ids_hbm, wte_hbm, wpe_hbm, out_hbm, idx_v, tok_v, pos_v, sem):
    core = lax.axis_index("c")
    sub = lax.axis_index("s")
    wid = sub * 2 + core
    pos_base = wid * ROWS_PER_W

    pltpu.sync_copy(wpe_hbm.at[pl.ds(pos_base, ROWS_PER_W)], pos_v)

    for b in range(BATCH):
        row_base = b * SEQ + pos_base
        pltpu.sync_copy(ids_hbm.at[pl.ds(row_base, ROWS_PER_W)], idx_v)
        pltpu.async_copy(wte_hbm.at[idx_v], tok_v, sem).wait()

        def add_row(r, carry):
            for k in range(VECS_PER_ROW):
                sl = pl.ds(k * LANES, LANES)
                tok_v[r, sl] = tok_v[r, sl] + pos_v[r, sl]
            return carry

        lax.fori_loop(0, ROWS_PER_W, add_row, 0)

        pltpu.sync_copy(tok_v, out_hbm.at[pl.ds(row_base, ROWS_PER_W)])


_emb = functools.partial(
    pl.kernel,
    mesh=plsc.VectorSubcoreMesh(core_axis_name="c", subcore_axis_name="s"),
    out_type=jax.ShapeDtypeStruct((NTOK, N_EMBD), jnp.float32),
    scratch_types=[
        pltpu.VMEM((ROWS_PER_W,), jnp.int32),
        pltpu.VMEM((ROWS_PER_W, N_EMBD), jnp.float32),
        pltpu.VMEM((ROWS_PER_W, N_EMBD), jnp.float32),
        pltpu.SemaphoreType.DMA,
    ],
)(_emb_body)


@jax.jit
def kernel(input_ids, wte, wpe):
    ids_flat = input_ids.reshape(-1).astype(jnp.int32)
    out = _emb(ids_flat, wte, wpe)
    return out.reshape(BATCH, SEQ, N_EMBD)
